# Initial kernel scaffold; baseline (speedup 1.0000x reference)
import jax
import jax.numpy as jnp
from jax.experimental import pallas as pl


def kernel(x, projection):
    raise NotImplementedError("write your pallas kernel here")


# speedup vs baseline: 16.8320x; 16.8320x over previous
import jax
import jax.numpy as jnp
import numpy as np
from jax.experimental import pallas as pl

_INPUT_DIM = 768
_EXP_DIM = 8192
_POOL = 30
_SEL = 8
_KEEP = 0.05
_K = int(_EXP_DIM * _KEEP)
_CHUNK = _EXP_DIM // _SEL
_ROWS = 128

_IMIN = np.int32(-2**31)


def _select_kernel(x_ref, p_ref, o_ref):
    x = x_ref[...]
    p = p_ref[...]
    e = jax.lax.dot_general(
        x, p, (((1,), (1,)), ((), ())),
        preferred_element_type=jnp.float32)

    b = jax.lax.bitcast_convert_type(e, jnp.int32)
    key = b ^ (jax.lax.shift_right_arithmetic(b, 31) & np.int32(0x7FFFFFFF))
    u = key ^ _IMIN

    R = x.shape[0]
    t = jnp.zeros((R, 1), jnp.int32)
    kp = jnp.full((R, 1), _K, jnp.int32)
    for i in range(32):
        sh = 31 - i
        bit = (1 << sh) if sh < 31 else int(_IMIN)
        cand = jax.lax.shift_right_logical(t, sh) | 1
        match = jax.lax.shift_right_logical(u, sh) == cand
        cnt = jnp.sum(match.astype(jnp.int32), axis=1, keepdims=True)
        ge = cnt >= kp
        t = jnp.where(ge, t | np.int32(bit), t)
        kp = jnp.where(ge, kp, kp - cnt)

    ts = t ^ _IMIN
    gt = key > ts
    eq = key == ts
    idx = jax.lax.broadcasted_iota(jnp.int32, (R, _EXP_DIM), 1)
    eqi = eq.astype(jnp.int32)
    tau = jnp.zeros((R, 1), jnp.int32)
    for bb in range(12, -1, -1):
        cand = tau + (1 << bb)
        c = jnp.sum(jnp.where(idx < cand, eqi, 0), axis=1, keepdims=True)
        tau = jnp.where(c < kp, cand, tau)
    keep = gt | (eq & (idx <= tau))

    o_ref[...] = jnp.where(keep, e, 0.0)


@jax.jit
def kernel(x, projection):
    B = x.shape[0]
    sparse_code = pl.pallas_call(
        _select_kernel,
        grid=(B // _ROWS,),
        in_specs=[
            pl.BlockSpec((_ROWS, _INPUT_DIM), lambda i: (i, 0)),
            pl.BlockSpec((_EXP_DIM, _INPUT_DIM), lambda i: (0, 0)),
        ],
        out_specs=pl.BlockSpec((_ROWS, _EXP_DIM), lambda i: (i, 0)),
        out_shape=jax.ShapeDtypeStruct((B, _EXP_DIM), jnp.float32),
    )(x, projection)

    indices = []
    for i in range(_SEL):
        start = i * _CHUNK
        chunk = sparse_code[:, start:start + _CHUNK]
        weights = jnp.arange(1, _CHUNK + 1, dtype=jnp.float32)
        weights = (weights * 2654435761.0) % _POOL
        hash_values = (chunk * weights[None, :]).sum(axis=1)
        prompt_idx = jnp.mod(hash_values, _POOL).astype(jnp.int32)
        indices.append(prompt_idx)
    return jnp.stack(indices, axis=1)


# speedup vs baseline: 19.0390x; 1.1311x over previous
import jax
import jax.numpy as jnp
import numpy as np
from jax.experimental import pallas as pl

_INPUT_DIM = 768
_EXP_DIM = 8192
_POOL = 30
_SEL = 8
_KEEP = 0.05
_K = int(_EXP_DIM * _KEEP)
_CHUNK = _EXP_DIM // _SEL
_ROWS = 128

_IMIN = np.int32(-2**31)


def _select_kernel(x_ref, p_ref, o_ref):
    x = x_ref[...]
    p = p_ref[...]
    e = jax.lax.dot_general(
        x, p, (((1,), (1,)), ((), ())),
        preferred_element_type=jnp.float32)

    b = jax.lax.bitcast_convert_type(e, jnp.int32)
    key = b ^ (jax.lax.shift_right_arithmetic(b, 31) & np.int32(0x7FFFFFFF))

    R = x.shape[0]
    t_u = jnp.zeros((R, 1), jnp.int32)
    for i in range(32):
        sh = 31 - i
        bit = np.int32((1 << sh) if sh < 31 else int(_IMIN))
        cand_u = t_u | bit
        cand_s = cand_u ^ _IMIN
        cnt = jnp.sum((key >= cand_s).astype(jnp.int32), axis=1, keepdims=True)
        t_u = jnp.where(cnt >= _K, cand_u, t_u)

    ts = t_u ^ _IMIN
    gt = key > ts
    eq = key == ts
    cnt_gt = jnp.sum(gt.astype(jnp.int32), axis=1, keepdims=True)
    kp = _K - cnt_gt

    idx = jax.lax.broadcasted_iota(jnp.int32, (R, _EXP_DIM), 1)
    eqi = eq.astype(jnp.int32)
    tau = jnp.zeros((R, 1), jnp.int32)
    for bb in range(12, -1, -1):
        cand = tau + (1 << bb)
        c = jnp.sum(jnp.where(idx < cand, eqi, 0), axis=1, keepdims=True)
        tau = jnp.where(c < kp, cand, tau)
    keep = gt | (eq & (idx <= tau))

    o_ref[...] = jnp.where(keep, e, 0.0)


@jax.jit
def kernel(x, projection):
    B = x.shape[0]
    sparse_code = pl.pallas_call(
        _select_kernel,
        grid=(B // _ROWS,),
        in_specs=[
            pl.BlockSpec((_ROWS, _INPUT_DIM), lambda i: (i, 0)),
            pl.BlockSpec((_EXP_DIM, _INPUT_DIM), lambda i: (0, 0)),
        ],
        out_specs=pl.BlockSpec((_ROWS, _EXP_DIM), lambda i: (i, 0)),
        out_shape=jax.ShapeDtypeStruct((B, _EXP_DIM), jnp.float32),
    )(x, projection)

    indices = []
    for i in range(_SEL):
        start = i * _CHUNK
        chunk = sparse_code[:, start:start + _CHUNK]
        weights = jnp.arange(1, _CHUNK + 1, dtype=jnp.float32)
        weights = (weights * 2654435761.0) % _POOL
        hash_values = (chunk * weights[None, :]).sum(axis=1)
        prompt_idx = jnp.mod(hash_values, _POOL).astype(jnp.int32)
        indices.append(prompt_idx)
    return jnp.stack(indices, axis=1)
